# Initial kernel scaffold; baseline (speedup 1.0000x reference)
import jax
import jax.numpy as jnp
from jax.experimental import pallas as pl


def kernel(x, edge_index, edge_weight, W, b, proj_W, proj_b):
    raise NotImplementedError("write your pallas kernel here")


# speedup vs baseline: 199.2009x; 199.2009x over previous
import functools

import jax
import jax.numpy as jnp
from jax import lax
from jax.experimental import pallas as pl
from jax.experimental.pallas import tpu as pltpu
from jax.experimental.pallas import tpu_sc as plsc

F32 = jnp.float32
I32 = jnp.int32

B, N, T, E = 8, 50000, 12, 800000
HORIZON = 12

NSC = 2
NTILE = 16
NW = NSC * NTILE
CH = 200
EPT = CH * 128
EP = EPT * NW
NP = 50176
RPS = NP // NTILE

_mesh = plsc.VectorSubcoreMesh(core_axis_name="c", subcore_axis_name="s")
_sc_params = pltpu.CompilerParams(use_tc_tiling_on_sc=False)


@functools.partial(
    pl.kernel,
    out_type=(
        jax.ShapeDtypeStruct((NP,), F32),
        jax.ShapeDtypeStruct((NP,), F32),
        jax.ShapeDtypeStruct((NP,), F32),
        jax.ShapeDtypeStruct((NP,), F32),
    ),
    mesh=_mesh,
    scratch_types=[
        pltpu.VMEM((CH, 128), I32),
        pltpu.VMEM((CH, 128), I32),
        pltpu.VMEM((CH, 128), F32),
        pltpu.VMEM((128,), F32),
        pltpu.VMEM((RPS,), F32),
        pltpu.VMEM_SHARED((NP,), F32),
        pltpu.VMEM_SHARED((NP,), F32),
    ],
    compiler_params=_sc_params,
)
def _degrees(row2, col2, w2, ones_h, z1_h, degw0, degw1, cnt0, cnt1,
             rowb, colb, wb, onesv, bounce, degw_s, cnt_s):
    cid = lax.axis_index("c")
    sid = lax.axis_index("s")
    wid = sid * NSC + cid
    off = sid * RPS
    pltpu.sync_copy(z1_h, bounce)
    pltpu.sync_copy(bounce, degw_s.at[pl.ds(off, RPS)])
    pltpu.sync_copy(bounce, cnt_s.at[pl.ds(off, RPS)])
    pltpu.sync_copy(ones_h, onesv)
    base = wid * CH
    pltpu.sync_copy(row2.at[pl.ds(base, CH)], rowb)
    pltpu.sync_copy(col2.at[pl.ds(base, CH)], colb)
    pltpu.sync_copy(w2.at[pl.ds(base, CH)], wb)
    plsc.subcore_barrier()

    def body(j, carry):
        pltpu.sync_copy(wb.at[j], degw_s.at[rowb.at[j]], add=True)
        pltpu.sync_copy(onesv, cnt_s.at[colb.at[j]], add=True)
        return carry

    lax.fori_loop(0, CH, body, 0)
    plsc.subcore_barrier()
    pltpu.sync_copy(degw_s.at[pl.ds(off, RPS)], bounce)

    @pl.when(cid == 0)
    def _():
        pltpu.sync_copy(bounce, degw0.at[pl.ds(off, RPS)])

    @pl.when(cid == 1)
    def _():
        pltpu.sync_copy(bounce, degw1.at[pl.ds(off, RPS)])

    pltpu.sync_copy(cnt_s.at[pl.ds(off, RPS)], bounce)

    @pl.when(cid == 0)
    def _():
        pltpu.sync_copy(bounce, cnt0.at[pl.ds(off, RPS)])

    @pl.when(cid == 1)
    def _():
        pltpu.sync_copy(bounce, cnt1.at[pl.ds(off, RPS)])


NB = 1024
GRID_N = NP // NB


def _prep_body(x_ref, dwa, dwb, ca, cb, g_ref, dinv_ref):
    xm = jnp.sum(x_ref[...], axis=2) * (1.0 / T)
    iso = 1.0 / (dwa[...] + dwb[...] + 1e-3)
    dinv = lax.rsqrt(ca[...] + cb[...] + 1.0)
    dinv_ref[...] = dinv
    eye = jnp.eye(B, dtype=F32)
    xmt = lax.dot_general(xm, eye, (((0,), (0,)), ((), ())),
                          preferred_element_type=F32)
    g_ref[...] = xmt * (iso * dinv)[:, None]


_prep = pl.pallas_call(
    _prep_body,
    grid=(GRID_N,),
    in_specs=[
        pl.BlockSpec((B, NB, T), lambda i: (0, i, 0)),
        pl.BlockSpec((NB,), lambda i: (i,)),
        pl.BlockSpec((NB,), lambda i: (i,)),
        pl.BlockSpec((NB,), lambda i: (i,)),
        pl.BlockSpec((NB,), lambda i: (i,)),
    ],
    out_specs=[
        pl.BlockSpec((NB, B), lambda i: (i, 0)),
        pl.BlockSpec((NB,), lambda i: (i,)),
    ],
    out_shape=(
        jax.ShapeDtypeStruct((NP, B), F32),
        jax.ShapeDtypeStruct((NP,), F32),
    ),
)


@functools.partial(
    pl.kernel,
    out_type=(
        jax.ShapeDtypeStruct((NP, B), F32),
        jax.ShapeDtypeStruct((NP, B), F32),
    ),
    mesh=_mesh,
    scratch_types=[
        pltpu.VMEM((CH, 128), I32),
        pltpu.VMEM((CH, 128), I32),
        pltpu.VMEM((128, B), F32),
        pltpu.VMEM((RPS, B), F32),
        pltpu.VMEM_SHARED((NP, B), F32),
        pltpu.VMEM_SHARED((NP, B), F32),
    ],
    compiler_params=_sc_params,
)
def _edgepass(row2, col2, g_h, z8_h, acc0, acc1,
              rowb, colb, gath, bounce, g_s, acc_s):
    cid = lax.axis_index("c")
    sid = lax.axis_index("s")
    wid = sid * NSC + cid
    off = sid * RPS
    pltpu.sync_copy(z8_h, bounce)
    pltpu.sync_copy(bounce, acc_s.at[pl.ds(off, RPS)])
    pltpu.sync_copy(g_h.at[pl.ds(off, RPS)], bounce)
    pltpu.sync_copy(bounce, g_s.at[pl.ds(off, RPS)])
    base = wid * CH
    pltpu.sync_copy(row2.at[pl.ds(base, CH)], rowb)
    pltpu.sync_copy(col2.at[pl.ds(base, CH)], colb)
    plsc.subcore_barrier()

    def body(j, carry):
        pltpu.sync_copy(g_s.at[rowb.at[j]], gath)
        pltpu.sync_copy(gath, acc_s.at[colb.at[j]], add=True)
        return carry

    lax.fori_loop(0, CH, body, 0)
    plsc.subcore_barrier()
    pltpu.sync_copy(acc_s.at[pl.ds(off, RPS)], bounce)

    @pl.when(cid == 0)
    def _():
        pltpu.sync_copy(bounce, acc0.at[pl.ds(off, RPS)])

    @pl.when(cid == 1)
    def _():
        pltpu.sync_copy(bounce, acc1.at[pl.ds(off, RPS)])


def _fin_body(acca, accb, g_ref, dinv_ref, w_ref, b_ref, pw_ref, pb_ref,
              out_ref):
    t = (acca[...] + accb[...] + g_ref[...]) * dinv_ref[...][:, None]
    eye = jnp.eye(B, dtype=F32)
    tt = lax.dot_general(eye, t, (((1,), (1,)), ((), ())),
                         preferred_element_type=F32)
    v = jnp.dot(w_ref[0, :], pw_ref[...], preferred_element_type=F32)
    c = jnp.dot(b_ref[...], pw_ref[...], preferred_element_type=F32) + pb_ref[...]
    out_ref[...] = tt[:, :, None] * v[None, None, :] + c[None, None, :]


_fin = pl.pallas_call(
    _fin_body,
    grid=(GRID_N,),
    in_specs=[
        pl.BlockSpec((NB, B), lambda i: (i, 0)),
        pl.BlockSpec((NB, B), lambda i: (i, 0)),
        pl.BlockSpec((NB, B), lambda i: (i, 0)),
        pl.BlockSpec((NB,), lambda i: (i,)),
        pl.BlockSpec((1, 64), lambda i: (0, 0)),
        pl.BlockSpec((64,), lambda i: (0,)),
        pl.BlockSpec((64, HORIZON), lambda i: (0, 0)),
        pl.BlockSpec((HORIZON,), lambda i: (0,)),
    ],
    out_specs=pl.BlockSpec((B, NB, HORIZON), lambda i: (0, i, 0)),
    out_shape=jax.ShapeDtypeStruct((B, N, HORIZON), F32),
)


def kernel(x, edge_index, edge_weight, W, b, proj_W, proj_b):
    x3 = x[..., 0]
    row = edge_index[0]
    col = edge_index[1]
    pad = EP - E
    rowp = jnp.concatenate([row, jnp.zeros((pad,), I32)])
    colp = jnp.concatenate([col, jnp.full((pad,), N, I32)])
    wp = jnp.concatenate([edge_weight, jnp.zeros((pad,), F32)])
    row2 = rowp.reshape(EP // 128, 128)
    col2 = colp.reshape(EP // 128, 128)
    w2 = wp.reshape(EP // 128, 128)
    ones_h = jnp.ones((128,), F32)
    z1 = jnp.zeros((RPS,), F32)
    z8 = jnp.zeros((RPS, B), F32)
    degw0, degw1, cnt0, cnt1 = _degrees(row2, col2, w2, ones_h, z1)
    g, dinv = _prep(x3, degw0, degw1, cnt0, cnt1)
    acc0, acc1 = _edgepass(row2, col2, g, z8)
    return _fin(acc0, acc1, g, dinv, W, b, proj_W, proj_b)


# speedup vs baseline: 210.5367x; 1.0569x over previous
import functools

import jax
import jax.numpy as jnp
from jax import lax
from jax.experimental import pallas as pl
from jax.experimental.pallas import tpu as pltpu
from jax.experimental.pallas import tpu_sc as plsc

F32 = jnp.float32
I32 = jnp.int32

B, N, T, E = 8, 50000, 12, 800000
HORIZON = 12

NSC = 2
NTILE = 16
NW = NSC * NTILE
ER = E // 128
CHS = 196
NP = 50176
RPS = NP // NTILE
NPR = NP // 16

_mesh = plsc.VectorSubcoreMesh(core_axis_name="c", subcore_axis_name="s")
_sc_params = pltpu.CompilerParams(use_tc_tiling_on_sc=False)


def _worker_span(wid):
    start = wid * ER // NW
    nrows = (wid + 1) * ER // NW - start
    return start, nrows


@functools.partial(
    pl.kernel,
    out_type=(
        jax.ShapeDtypeStruct((NP,), F32),
        jax.ShapeDtypeStruct((NP,), F32),
        jax.ShapeDtypeStruct((NP,), F32),
        jax.ShapeDtypeStruct((NP,), F32),
    ),
    mesh=_mesh,
    scratch_types=[
        pltpu.VMEM((CHS, 128), I32),
        pltpu.VMEM((CHS, 128), I32),
        pltpu.VMEM((CHS, 128), F32),
        pltpu.VMEM((128,), F32),
        pltpu.VMEM((RPS,), F32),
        pltpu.VMEM_SHARED((NP,), F32),
        pltpu.VMEM_SHARED((NP,), F32),
    ],
    compiler_params=_sc_params,
)
def _degrees(row2, col2, w2, ones_h, z1_h, degw0, degw1, cnt0, cnt1,
             rowb, colb, wb, onesv, bounce, degw_s, cnt_s):
    cid = lax.axis_index("c")
    sid = lax.axis_index("s")
    wid = sid * NSC + cid
    off = sid * RPS
    pltpu.sync_copy(z1_h, bounce)
    pltpu.sync_copy(bounce, degw_s.at[pl.ds(off, RPS)])
    pltpu.sync_copy(bounce, cnt_s.at[pl.ds(off, RPS)])
    pltpu.sync_copy(ones_h, onesv)
    start, nrows = _worker_span(wid)
    pltpu.sync_copy(row2.at[pl.ds(start, CHS)], rowb)
    pltpu.sync_copy(col2.at[pl.ds(start, CHS)], colb)
    pltpu.sync_copy(w2.at[pl.ds(start, CHS)], wb)
    plsc.subcore_barrier()

    def body(j, carry):
        pltpu.sync_copy(wb.at[j], degw_s.at[rowb.at[j]], add=True)
        pltpu.sync_copy(onesv, cnt_s.at[colb.at[j]], add=True)
        return carry

    lax.fori_loop(0, nrows, body, 0)
    plsc.subcore_barrier()
    pltpu.sync_copy(degw_s.at[pl.ds(off, RPS)], bounce)

    @pl.when(cid == 0)
    def _():
        pltpu.sync_copy(bounce, degw0.at[pl.ds(off, RPS)])

    @pl.when(cid == 1)
    def _():
        pltpu.sync_copy(bounce, degw1.at[pl.ds(off, RPS)])

    pltpu.sync_copy(cnt_s.at[pl.ds(off, RPS)], bounce)

    @pl.when(cid == 0)
    def _():
        pltpu.sync_copy(bounce, cnt0.at[pl.ds(off, RPS)])

    @pl.when(cid == 1)
    def _():
        pltpu.sync_copy(bounce, cnt1.at[pl.ds(off, RPS)])


NB = 1024
GRID_N = NP // NB


def _xmean_body(x_ref, xm_ref):
    xm_ref[...] = jnp.sum(x_ref[...], axis=2) * (1.0 / T)


_xmean = pl.pallas_call(
    _xmean_body,
    grid=(GRID_N,),
    in_specs=[pl.BlockSpec((B, NB, T), lambda i: (0, i, 0))],
    out_specs=pl.BlockSpec((B, NB), lambda i: (0, i)),
    out_shape=jax.ShapeDtypeStruct((B, NP), F32),
)


def _scale_body(xm_ref, dwa, dwb, ca, cb, xs_ref, dinv_ref):
    iso = 1.0 / (dwa[...] + dwb[...] + 1e-3)
    dinv = lax.rsqrt(ca[...] + cb[...] + 1.0)
    dinv_ref[...] = dinv
    xs_ref[...] = xm_ref[...] * (iso * dinv)[None, :]


_scale = pl.pallas_call(
    _scale_body,
    grid=(GRID_N,),
    in_specs=[pl.BlockSpec((B, NB), lambda i: (0, i))]
    + [pl.BlockSpec((NB,), lambda i: (i,))] * 4,
    out_specs=[
        pl.BlockSpec((B, NB), lambda i: (0, i)),
        pl.BlockSpec((NB,), lambda i: (i,)),
    ],
    out_shape=(
        jax.ShapeDtypeStruct((B, NP), F32),
        jax.ShapeDtypeStruct((NP,), F32),
    ),
)


@functools.partial(
    pl.kernel,
    out_type=(
        jax.ShapeDtypeStruct((NP, B), F32),
        jax.ShapeDtypeStruct((NP, B), F32),
    ),
    mesh=_mesh,
    scratch_types=[
        pltpu.VMEM((CHS, 128), I32),
        pltpu.VMEM((CHS, 128), I32),
        pltpu.VMEM((RPS, B), F32),
        pltpu.VMEM((128, B), F32),
        pltpu.VMEM_SHARED((NP, B), F32),
        pltpu.VMEM_SHARED((NP, B), F32),
    ],
    compiler_params=_sc_params,
)
def _edgepass(row2, col2, g_h, z8_h, acc0, acc1,
              rowb, colb, gbuf, gath, g_s, acc_s):
    cid = lax.axis_index("c")
    sid = lax.axis_index("s")
    wid = sid * NSC + cid
    off = sid * RPS
    start, nrows = _worker_span(wid)
    pltpu.sync_copy(row2.at[pl.ds(start, CHS)], rowb)
    pltpu.sync_copy(col2.at[pl.ds(start, CHS)], colb)
    pltpu.sync_copy(g_h.at[pl.ds(off, RPS)], gbuf)
    pltpu.sync_copy(gbuf, g_s.at[pl.ds(off, RPS)])

    @pl.when(cid == 0)
    def _():
        pltpu.sync_copy(gbuf, acc_s.at[pl.ds(off, RPS)])

    @pl.when(cid == 1)
    def _():
        pltpu.sync_copy(z8_h, gbuf)
        pltpu.sync_copy(gbuf, acc_s.at[pl.ds(off, RPS)])

    plsc.subcore_barrier()

    def ebody(j, carry):
        pltpu.sync_copy(g_s.at[rowb.at[j]], gath)
        pltpu.sync_copy(gath, acc_s.at[colb.at[j]], add=True)
        return carry

    lax.fori_loop(0, nrows, ebody, 0)
    plsc.subcore_barrier()
    pltpu.sync_copy(acc_s.at[pl.ds(off, RPS)], gbuf)

    @pl.when(cid == 0)
    def _():
        pltpu.sync_copy(gbuf, acc0.at[pl.ds(off, RPS)])

    @pl.when(cid == 1)
    def _():
        pltpu.sync_copy(gbuf, acc1.at[pl.ds(off, RPS)])


def _fin_body(ta, tb, dinv_ref, w_ref, b_ref, pw_ref, pb_ref, out_ref):
    tt = (ta[...] + tb[...]) * dinv_ref[...][None, :]
    v = jnp.dot(w_ref[0, :], pw_ref[...], preferred_element_type=F32)
    c = jnp.dot(b_ref[...], pw_ref[...], preferred_element_type=F32) + pb_ref[...]
    out_ref[...] = tt[:, :, None] * v[None, None, :] + c[None, None, :]


_fin = pl.pallas_call(
    _fin_body,
    grid=(GRID_N,),
    in_specs=[
        pl.BlockSpec((B, NB), lambda i: (0, i)),
        pl.BlockSpec((B, NB), lambda i: (0, i)),
        pl.BlockSpec((NB,), lambda i: (i,)),
        pl.BlockSpec((1, 64), lambda i: (0, 0)),
        pl.BlockSpec((64,), lambda i: (0,)),
        pl.BlockSpec((64, HORIZON), lambda i: (0, 0)),
        pl.BlockSpec((HORIZON,), lambda i: (0,)),
    ],
    out_specs=pl.BlockSpec((B, NB, HORIZON), lambda i: (0, i, 0)),
    out_shape=jax.ShapeDtypeStruct((B, N, HORIZON), F32),
)


def kernel(x, edge_index, edge_weight, W, b, proj_W, proj_b):
    x3 = x[..., 0]
    row2 = edge_index[0].reshape(ER, 128)
    col2 = edge_index[1].reshape(ER, 128)
    w2 = edge_weight.reshape(ER, 128)
    ones_h = jnp.ones((128,), F32)
    z1 = jnp.zeros((RPS,), F32)
    z8 = jnp.zeros((RPS, B), F32)
    degw0, degw1, cnt0, cnt1 = _degrees(row2, col2, w2, ones_h, z1)
    xm = _xmean(x3)
    xs, dinv = _scale(xm, degw0, degw1, cnt0, cnt1)
    g = jnp.transpose(xs)
    acc0, acc1 = _edgepass(row2, col2, g, z8)
    a0 = jnp.transpose(acc0)
    a1 = jnp.transpose(acc1)
    return _fin(a0, a1, dinv, W, b, proj_W, proj_b)
